# Initial kernel scaffold; baseline (speedup 1.0000x reference)
import jax
import jax.numpy as jnp
from jax import lax
from jax.experimental import pallas as pl
from jax.experimental.pallas import tpu as pltpu
from jax.experimental.pallas import tpu_sc as plsc

VOCAB_SIZE = 1_000_000
EMBED_DIM = 32
BATCH = 16384
HIST = 50
B_TOTAL = BATCH * HIST
NUM_WORKERS = 32
B_PER_W = B_TOTAL // NUM_WORKERS
CHUNK = 3200
NCHUNK = B_PER_W // CHUNK


def _body(idx_hbm, table_hbm, out_hbm, idx_v, rows_v, sem):
    wid = lax.axis_index("s") * 2 + lax.axis_index("c")
    base = wid * B_PER_W

    @pl.loop(0, NCHUNK)
    def _chunk(c):
        off = base + c * CHUNK
        pltpu.sync_copy(idx_hbm.at[pl.ds(off, CHUNK)], idx_v)
        pltpu.async_copy(table_hbm.at[idx_v], rows_v, sem).wait()
        pltpu.sync_copy(rows_v, out_hbm.at[pl.ds(off, CHUNK)])


@jax.jit
def _embed(token_ids_flat, embeddings):
    mesh = plsc.VectorSubcoreMesh(core_axis_name="c", subcore_axis_name="s")
    grid_kernel = pl.kernel(
        _body,
        out_type=jax.ShapeDtypeStruct((B_TOTAL, EMBED_DIM), jnp.float32),
        mesh=mesh,
        scratch_types=[
            pltpu.VMEM((CHUNK,), jnp.int32),
            pltpu.VMEM((CHUNK, EMBED_DIM), jnp.float32),
            pltpu.SemaphoreType.DMA,
        ],
    )
    return grid_kernel(token_ids_flat, embeddings)


def kernel(token_ids, embeddings):
    flat = token_ids.reshape(-1).astype(jnp.int32)
    out = _embed(flat, embeddings)
    return out.reshape(BATCH, HIST, EMBED_DIM)


# speedup vs baseline: 1.1102x; 1.1102x over previous
import jax
import jax.numpy as jnp
from jax import lax
from jax.experimental import pallas as pl
from jax.experimental.pallas import tpu as pltpu
from jax.experimental.pallas import tpu_sc as plsc

VOCAB_SIZE = 1_000_000
EMBED_DIM = 32
BATCH = 16384
HIST = 50
B_TOTAL = BATCH * HIST
NUM_WORKERS = 32
B_PER_W = B_TOTAL // NUM_WORKERS
CHUNK = 3200
NCHUNK = B_PER_W // CHUNK


def _body(idx_hbm, table_hbm, out_hbm, idx_v, rows_v, sem):
    wid = lax.axis_index("s") * 2 + lax.axis_index("c")
    base = wid * B_PER_W

    @pl.loop(0, NCHUNK)
    def _chunk(c):
        off = base + c * CHUNK
        pltpu.sync_copy(idx_hbm.at[pl.ds(off, CHUNK)], idx_v)
        pltpu.async_copy(table_hbm.at[idx_v], rows_v, sem).wait()
        pltpu.sync_copy(rows_v, out_hbm.at[pl.ds(off, CHUNK)])


@jax.jit
def _embed(token_ids_flat, embeddings):
    mesh = plsc.VectorSubcoreMesh(core_axis_name="c", subcore_axis_name="s")
    grid_kernel = pl.kernel(
        _body,
        out_type=jax.ShapeDtypeStruct((B_TOTAL, EMBED_DIM), jnp.float32),
        mesh=mesh,
        scratch_types=[
            pltpu.VMEM((CHUNK,), jnp.int32),
            pltpu.VMEM((CHUNK, EMBED_DIM), jnp.float32),
            pltpu.SemaphoreType.DMA,
        ],
        compiler_params=pltpu.CompilerParams(use_tc_tiling_on_sc=False),
    )
    return grid_kernel(token_ids_flat, embeddings)


def kernel(token_ids, embeddings):
    flat = token_ids.reshape(-1).astype(jnp.int32)
    out = _embed(flat, embeddings)
    return out.reshape(BATCH, HIST, EMBED_DIM)


# speedup vs baseline: 1.7518x; 1.5779x over previous
import jax
import jax.numpy as jnp
from jax import lax
from jax.experimental import pallas as pl
from jax.experimental.pallas import tpu as pltpu
from jax.experimental.pallas import tpu_sc as plsc

VOCAB_SIZE = 1_000_000
EMBED_DIM = 32
BATCH = 16384
HIST = 50
HIST_PAD = 56
LANE_PAD = 128
Q_TOTAL = BATCH * HIST_PAD
NUM_WORKERS = 32
Q_PER_W = Q_TOTAL // NUM_WORKERS
CQ = 3584
NCHUNK = Q_PER_W // CQ


def _body(idx_hbm, table_hbm, out_hbm, idx_v, rows_v, sem):
    wid = lax.axis_index("s") * 2 + lax.axis_index("c")
    base = wid * Q_PER_W

    @pl.loop(0, NCHUNK)
    def _chunk(c):
        q0 = base + c * CQ
        pltpu.sync_copy(idx_hbm.at[pl.ds(q0, CQ)], idx_v)
        pltpu.async_copy(table_hbm.at[idx_v], rows_v, sem).wait()
        pltpu.sync_copy(rows_v, out_hbm.at[pl.ds(q0, CQ), pl.ds(0, EMBED_DIM)])


@jax.jit
def _embed(token_ids, embeddings):
    idx56 = jnp.concatenate(
        [token_ids, token_ids[:, : HIST_PAD - HIST]], axis=1
    ).astype(jnp.int32)
    idx_flat = idx56.reshape(-1)
    table = jnp.minimum(embeddings, jnp.float32(3.4e38))

    mesh = plsc.VectorSubcoreMesh(core_axis_name="c", subcore_axis_name="s")
    grid_kernel = pl.kernel(
        _body,
        out_type=jax.ShapeDtypeStruct((Q_TOTAL, LANE_PAD), jnp.float32),
        mesh=mesh,
        scratch_types=[
            pltpu.VMEM((CQ,), jnp.int32),
            pltpu.VMEM((CQ, EMBED_DIM), jnp.float32),
            pltpu.SemaphoreType.DMA,
        ],
        compiler_params=pltpu.CompilerParams(use_tc_tiling_on_sc=False),
    )
    padded = grid_kernel(idx_flat, table)
    padded3 = padded.reshape(BATCH, HIST_PAD, LANE_PAD)
    return lax.slice(padded3, (0, 0, 0), (BATCH, HIST, EMBED_DIM))


def kernel(token_ids, embeddings):
    return _embed(token_ids, embeddings)


# speedup vs baseline: 2.5090x; 1.4323x over previous
import jax
import jax.numpy as jnp
from jax import lax
from jax.experimental import pallas as pl
from jax.experimental.pallas import tpu as pltpu
from jax.experimental.pallas import tpu_sc as plsc

VOCAB_SIZE = 1_000_000
EMBED_DIM = 32
BATCH = 16384
HIST = 50
HIST_PAD = 56
LANE_PAD = 128
Q_TOTAL = BATCH * HIST_PAD
NUM_WORKERS = 32
Q_PER_W = Q_TOTAL // NUM_WORKERS
CQ = 3584
NCHUNK = Q_PER_W // CQ


def _body(idx_hbm, table_hbm, out_hbm, idx_v, rows_v, sem):
    wid = lax.axis_index("s") * 2 + lax.axis_index("c")
    base = wid * Q_PER_W

    @pl.loop(0, NCHUNK)
    def _chunk(c):
        q0 = base + c * CQ
        pltpu.sync_copy(idx_hbm.at[pl.ds(q0, CQ)], idx_v)
        pltpu.async_copy(table_hbm.at[idx_v], rows_v, sem).wait()
        pltpu.sync_copy(rows_v, out_hbm.at[pl.ds(q0, CQ), pl.ds(0, EMBED_DIM)])


@jax.jit
def _embed(token_ids, embeddings):
    idx56 = jnp.concatenate(
        [token_ids, token_ids[:, : HIST_PAD - HIST]], axis=1
    ).astype(jnp.int32)
    idx_flat = idx56.reshape(-1)

    mesh = plsc.VectorSubcoreMesh(core_axis_name="c", subcore_axis_name="s")
    grid_kernel = pl.kernel(
        _body,
        out_type=jax.ShapeDtypeStruct((Q_TOTAL, LANE_PAD), jnp.float32),
        mesh=mesh,
        scratch_types=[
            pltpu.VMEM((CQ,), jnp.int32),
            pltpu.VMEM((CQ, EMBED_DIM), jnp.float32),
            pltpu.SemaphoreType.DMA,
        ],
        compiler_params=pltpu.CompilerParams(use_tc_tiling_on_sc=False),
    )
    padded = grid_kernel(idx_flat, embeddings)
    padded3 = padded.reshape(BATCH, HIST_PAD, LANE_PAD)
    return lax.slice(padded3, (0, 0, 0), (BATCH, HIST, EMBED_DIM))


def kernel(token_ids, embeddings):
    return _embed(token_ids, embeddings)
